# baseline (speedup 1.0000x reference)
import functools

import jax
import jax.numpy as jnp
from jax import lax
from jax.experimental import pallas as pl
from jax.experimental.pallas import tpu as pltpu
from jax.experimental.pallas import tpu_sc as plsc

NC = 2
NS = 16
LANES = 16
NW = NC * NS

EC = 16
IDX_CHUNK = 128

TB = 128


def _tr_body(vocab, emb, wt_hid, wt_out, dh_flat, do_flat,
             ih0, io0, ih1, io1, obh0, obo0, obh1, obo1,
             sem0, sem1, semw0, semw1):
    nfull = vocab // TB
    per = nfull // NW + 1
    if per % 2:
        per += 1
    wid = lax.axis_index("s") * NC + lax.axis_index("c")
    lanes_e = lax.iota(jnp.int32, 16) * emb
    blk = TB * emb

    def fire(b, ih, io, sem):
        pltpu.async_copy(wt_hid.at[:, pl.ds(b * TB, TB)], ih, sem)
        pltpu.async_copy(wt_out.at[:, pl.ds(b * TB, TB)], io, sem)

    def drain(ih, io, sem):
        pltpu.make_async_copy(wt_hid.at[:, pl.ds(0, TB)], ih, sem).wait()
        pltpu.make_async_copy(wt_out.at[:, pl.ds(0, TB)], io, sem).wait()

    def transpose(ih, io, obh, obo):
        def dim(d, carry):
            for g in range(TB // 16):
                idx = lanes_e + (d + g * 16 * emb)
                plsc.store_scatter(obh, [idx], ih[d, pl.ds(g * 16, 16)])
                plsc.store_scatter(obo, [idx], io[d, pl.ds(g * 16, 16)])
            return carry
        lax.fori_loop(0, emb, dim, 0)

    def write(b, obh, obo, semw, n=None):
        n = blk if n is None else n
        pltpu.async_copy(obh.at[pl.ds(0, n)], dh_flat.at[pl.ds(b * blk, n)],
                         semw)
        pltpu.async_copy(obo.at[pl.ds(0, n)], do_flat.at[pl.ds(b * blk, n)],
                         semw)

    def wait_write(obh, obo, semw, n=None):
        n = blk if n is None else n
        pltpu.make_async_copy(obh.at[pl.ds(0, n)], dh_flat.at[pl.ds(0, n)],
                              semw).wait()
        pltpu.make_async_copy(obo.at[pl.ds(0, n)], do_flat.at[pl.ds(0, n)],
                              semw).wait()

    def bidx(i):
        return jnp.minimum(wid + NW * i, nfull - 1)

    fire(bidx(0), ih0, io0, sem0)
    fire(bidx(1), ih1, io1, sem1)
    drain(ih0, io0, sem0)
    transpose(ih0, io0, obh0, obo0)
    write(bidx(0), obh0, obo0, semw0)
    fire(bidx(2), ih0, io0, sem0)
    drain(ih1, io1, sem1)
    transpose(ih1, io1, obh1, obo1)
    write(bidx(1), obh1, obo1, semw1)

    def outer(p, carry):
        i0 = 2 * p
        fire(bidx(i0 + 1), ih1, io1, sem1)
        drain(ih0, io0, sem0)
        wait_write(obh0, obo0, semw0)
        transpose(ih0, io0, obh0, obo0)
        write(bidx(i0), obh0, obo0, semw0)
        fire(bidx(i0 + 2), ih0, io0, sem0)
        drain(ih1, io1, sem1)
        wait_write(obh1, obo1, semw1)
        transpose(ih1, io1, obh1, obo1)
        write(bidx(i0 + 1), obh1, obo1, semw1)
        return carry

    lax.fori_loop(1, per // 2, outer, 0)
    drain(ih0, io0, sem0)
    wait_write(obh0, obo0, semw0)
    wait_write(obh1, obo1, semw1)

    rem = vocab - nfull * TB
    if rem:
        @pl.when(wid == 0)
        def _tail():
            b0 = jnp.int32(nfull) + jnp.int32(0)
            fire(b0, ih0, io0, sem0)
            drain(ih0, io0, sem0)
            transpose(ih0, io0, obh0, obo0)
            write(b0, obh0, obo0, semw0, n=rem * emb)
            wait_write(obh0, obo0, semw0, n=rem * emb)


def _sc_body(pw, neg, emb, ix_hbm, ia_hbm, w_hid, w_out, pos_out, negs_out,
             xi_v, ia_v, x_v, rows0, rows1, pos_v, negs_v, semx, sem0, sem1):
    npe = neg + 1
    nseg = emb // LANES
    steps = pw // EC
    rows_n = EC * npe
    chunks = [(o, min(IDX_CHUNK, rows_n - o)) for o in range(0, rows_n, IDX_CHUNK)]
    wid = lax.axis_index("s") * NC + lax.axis_index("c")
    base = wid * pw
    lane0 = lax.iota(jnp.int32, 16) == 0

    def scatter1(ref, pos_i, val):
        plsc.store_scatter(ref, [jnp.broadcast_to(pos_i, (16,))],
                           jnp.broadcast_to(val, (16,)), mask=lane0)

    pltpu.sync_copy(ix_hbm.at[pl.ds(base, pw)], xi_v)
    pltpu.sync_copy(ia_hbm.at[pl.ds(base * npe, pw * npe)], ia_v)

    xcps = [pltpu.make_async_copy(
        w_hid.at[xi_v.at[pl.ds(j * IDX_CHUNK, IDX_CHUNK)]],
        x_v.at[pl.ds(j * IDX_CHUNK, IDX_CHUNK)], semx)
        for j in range(pw // IDX_CHUNK)]
    for cp in xcps:
        cp.start()

    def fire(s, rbuf, sem):
        off = s * rows_n
        for o, c in chunks:
            pltpu.async_copy(w_out.at[ia_v.at[pl.ds(off + o, c)]],
                             rbuf.at[pl.ds(o, c)], sem)

    def drain(rbuf, sem):
        for o, c in chunks:
            pltpu.make_async_copy(w_out.at[ia_v.at[pl.ds(o, c)]],
                                  rbuf.at[pl.ds(o, c)], sem).wait()

    fire(0, rows0, sem0)
    for cp in xcps:
        cp.wait()

    def compute(s, rbuf):
        def elem(e, carry):
            b = s * EC + e
            xr = [x_v[b, pl.ds(k * LANES, LANES)] for k in range(nseg)]
            r0 = e * npe
            yr = [rbuf[r0, pl.ds(k * LANES, LANES)] for k in range(nseg)]
            acc = xr[0] * yr[0]
            for k in range(1, nseg):
                acc = acc + xr[k] * yr[k]
            scatter1(pos_v, b, jnp.sum(acc))
            for n_i in range(neg):
                nr = [rbuf[r0 + 1 + n_i, pl.ds(k * LANES, LANES)]
                      for k in range(nseg)]
                nacc = nr[0] * xr[0]
                for k in range(1, nseg):
                    nacc = nacc + nr[k] * xr[k]
                scatter1(negs_v, b * neg + n_i, jnp.sum(nacc))
            return carry

        lax.fori_loop(0, EC, elem, 0)

    def outer(t, carry):
        s0 = 2 * t
        fire(s0 + 1, rows1, sem1)
        drain(rows0, sem0)
        compute(s0, rows0)
        fire(jnp.minimum(s0 + 2, steps - 1), rows0, sem0)
        drain(rows1, sem1)
        compute(s0 + 1, rows1)
        return carry

    lax.fori_loop(0, steps // 2, outer, 0)
    drain(rows0, sem0)

    pltpu.sync_copy(pos_v, pos_out.at[pl.ds(base, pw)])
    pltpu.sync_copy(negs_v, negs_out.at[pl.ds(base * neg, pw * neg)])


def _tc_loss_body(pos_ref, neg_ref, out_ref):
    p = pos_ref[...]
    n = neg_ref[...]
    ls_p = jnp.minimum(p, 0.0) - jnp.log1p(jnp.exp(-jnp.abs(p)))
    ls_n = jnp.minimum(-n, 0.0) - jnp.log1p(jnp.exp(-jnp.abs(n)))
    out_ref[0, 0] = -(jnp.sum(ls_p) + jnp.sum(ls_n))


def kernel(positive_pairs, negative_samples, W_hid, W_out):
    batch, neg = negative_samples.shape
    vocab, emb = W_hid.shape
    pw = batch // NW
    npe = neg + 1

    ix = positive_pairs[:, 0]
    ia = jnp.concatenate(
        [positive_pairs[:, 1:2], negative_samples], axis=1).reshape(-1)

    mesh = plsc.VectorSubcoreMesh(
        core_axis_name="c", subcore_axis_name="s",
        num_cores=NC, num_subcores=NS)

    transpose_tables = pl.kernel(
        functools.partial(_tr_body, vocab, emb),
        out_type=(jax.ShapeDtypeStruct((vocab * emb,), jnp.float32),
                  jax.ShapeDtypeStruct((vocab * emb,), jnp.float32)),
        mesh=mesh,
        scratch_types=[
            pltpu.VMEM((emb, TB), jnp.float32),
            pltpu.VMEM((emb, TB), jnp.float32),
            pltpu.VMEM((emb, TB), jnp.float32),
            pltpu.VMEM((emb, TB), jnp.float32),
            pltpu.VMEM((TB * emb,), jnp.float32),
            pltpu.VMEM((TB * emb,), jnp.float32),
            pltpu.VMEM((TB * emb,), jnp.float32),
            pltpu.VMEM((TB * emb,), jnp.float32),
            pltpu.SemaphoreType.DMA,
            pltpu.SemaphoreType.DMA,
            pltpu.SemaphoreType.DMA,
            pltpu.SemaphoreType.DMA,
        ],
        compiler_params=pltpu.CompilerParams(
            needs_layout_passes=False, use_tc_tiling_on_sc=True,
            disable_bounds_checks=True),
    )
    dh_flat, do_flat = transpose_tables(W_hid.T, W_out.T)
    w_hid_d = dh_flat.reshape(vocab, emb)
    w_out_d = do_flat.reshape(vocab, emb)
    sc_scores = pl.kernel(
        functools.partial(_sc_body, pw, neg, emb),
        out_type=(jax.ShapeDtypeStruct((batch,), jnp.float32),
                  jax.ShapeDtypeStruct((batch * neg,), jnp.float32)),
        mesh=mesh,
        scratch_types=[
            pltpu.VMEM((pw,), jnp.int32),
            pltpu.VMEM((pw * npe,), jnp.int32),
            pltpu.VMEM((pw, emb), jnp.float32),
            pltpu.VMEM((EC * npe, emb), jnp.float32),
            pltpu.VMEM((EC * npe, emb), jnp.float32),
            pltpu.VMEM((pw,), jnp.float32),
            pltpu.VMEM((pw * neg,), jnp.float32),
            pltpu.SemaphoreType.DMA,
            pltpu.SemaphoreType.DMA,
            pltpu.SemaphoreType.DMA,
        ],
        compiler_params=pltpu.CompilerParams(
            needs_layout_passes=False, use_tc_tiling_on_sc=False),
    )
    pos_s, neg_s = sc_scores(ix, ia, w_hid_d, w_out_d)

    pos2 = pos_s.reshape(batch // 128, 128)
    neg2 = neg_s.reshape(batch * neg // 128, 128)
    loss = pl.pallas_call(
        _tc_loss_body,
        out_shape=jax.ShapeDtypeStruct((1, 1), jnp.float32),
        out_specs=pl.BlockSpec(memory_space=pltpu.SMEM),
    )(pos2, neg2)
    return loss[0, 0]


# speedup vs baseline: 2.5611x; 2.5611x over previous
import functools

import jax
import jax.numpy as jnp
from jax import lax
from jax.experimental import pallas as pl
from jax.experimental.pallas import tpu as pltpu
from jax.experimental.pallas import tpu_sc as plsc

NC = 2
NS = 16
LANES = 16
NW = NC * NS

EC = 16
IDX_CHUNK = 128

TB = 128


def _tr_body(vocab, emb, wt_hid, wt_out, dh_flat, do_flat,
             ih0, io0, ih1, io1, obh0, obo0, obh1, obo1,
             sem0, sem1, semw0, semw1):
    nfull = vocab // TB
    per = nfull // NW + 1
    if per % 2:
        per += 1
    wid = lax.axis_index("s") * NC + lax.axis_index("c")
    blk = TB * emb

    def fire(b, ih, io, sem):
        pltpu.async_copy(wt_hid.at[:, pl.ds(b * TB, TB)], ih, sem)
        pltpu.async_copy(wt_out.at[:, pl.ds(b * TB, TB)], io, sem)

    def drain(ih, io, sem):
        pltpu.make_async_copy(wt_hid.at[:, pl.ds(0, TB)], ih, sem).wait()
        pltpu.make_async_copy(wt_out.at[:, pl.ds(0, TB)], io, sem).wait()

    def transpose(ih, io, obh, obo):
        lanes = lax.iota(jnp.int32, 16)

        def grp(g, carry):
            cols = lanes + g * 16
            colse = cols * emb
            for d0 in range(0, emb, 16):
                for i in range(16):
                    rows = ((lanes + i) & 15) + d0
                    v_h = plsc.load_gather(ih, [rows, cols])
                    v_o = plsc.load_gather(io, [rows, cols])
                    wr = colse + rows
                    plsc.store_scatter(obh, [wr], v_h)
                    plsc.store_scatter(obo, [wr], v_o)
            return carry

        lax.fori_loop(0, TB // 16, grp, 0)

    def write(b, obh, obo, semw, n=None):
        n = blk if n is None else n
        pltpu.async_copy(obh.at[pl.ds(0, n)], dh_flat.at[pl.ds(b * blk, n)],
                         semw)
        pltpu.async_copy(obo.at[pl.ds(0, n)], do_flat.at[pl.ds(b * blk, n)],
                         semw)

    def wait_write(obh, obo, semw, n=None):
        n = blk if n is None else n
        pltpu.make_async_copy(obh.at[pl.ds(0, n)], dh_flat.at[pl.ds(0, n)],
                              semw).wait()
        pltpu.make_async_copy(obo.at[pl.ds(0, n)], do_flat.at[pl.ds(0, n)],
                              semw).wait()

    def bidx(i):
        return jnp.minimum(wid + NW * i, nfull - 1)

    fire(bidx(0), ih0, io0, sem0)
    fire(bidx(1), ih1, io1, sem1)
    drain(ih0, io0, sem0)
    transpose(ih0, io0, obh0, obo0)
    write(bidx(0), obh0, obo0, semw0)
    fire(bidx(2), ih0, io0, sem0)
    drain(ih1, io1, sem1)
    transpose(ih1, io1, obh1, obo1)
    write(bidx(1), obh1, obo1, semw1)

    def outer(p, carry):
        i0 = 2 * p
        fire(bidx(i0 + 1), ih1, io1, sem1)
        drain(ih0, io0, sem0)
        wait_write(obh0, obo0, semw0)
        transpose(ih0, io0, obh0, obo0)
        write(bidx(i0), obh0, obo0, semw0)
        fire(bidx(i0 + 2), ih0, io0, sem0)
        drain(ih1, io1, sem1)
        wait_write(obh1, obo1, semw1)
        transpose(ih1, io1, obh1, obo1)
        write(bidx(i0 + 1), obh1, obo1, semw1)
        return carry

    lax.fori_loop(1, per // 2, outer, 0)
    drain(ih0, io0, sem0)
    wait_write(obh0, obo0, semw0)
    wait_write(obh1, obo1, semw1)

    rem = vocab - nfull * TB
    if rem:
        @pl.when(wid == 0)
        def _tail():
            b0 = jnp.int32(nfull) + jnp.int32(0)
            fire(b0, ih0, io0, sem0)
            drain(ih0, io0, sem0)
            transpose(ih0, io0, obh0, obo0)
            write(b0, obh0, obo0, semw0, n=rem * emb)
            wait_write(obh0, obo0, semw0, n=rem * emb)


def _sc_body(pw, neg, emb, ix_hbm, ia_hbm, w_hid, w_out, pos_out, negs_out,
             xi_v, ia_v, x_v, rows0, rows1, pos_v, negs_v, semx, sem0, sem1):
    npe = neg + 1
    nseg = emb // LANES
    steps = pw // EC
    rows_n = EC * npe
    chunks = [(o, min(IDX_CHUNK, rows_n - o)) for o in range(0, rows_n, IDX_CHUNK)]
    wid = lax.axis_index("s") * NC + lax.axis_index("c")
    base = wid * pw
    lane0 = lax.iota(jnp.int32, 16) == 0

    def scatter1(ref, pos_i, val):
        plsc.store_scatter(ref, [jnp.broadcast_to(pos_i, (16,))],
                           jnp.broadcast_to(val, (16,)), mask=lane0)

    pltpu.sync_copy(ix_hbm.at[pl.ds(base, pw)], xi_v)
    pltpu.sync_copy(ia_hbm.at[pl.ds(base * npe, pw * npe)], ia_v)

    xcps = [pltpu.make_async_copy(
        w_hid.at[xi_v.at[pl.ds(j * IDX_CHUNK, IDX_CHUNK)]],
        x_v.at[pl.ds(j * IDX_CHUNK, IDX_CHUNK)], semx)
        for j in range(pw // IDX_CHUNK)]
    for cp in xcps:
        cp.start()

    def fire(s, rbuf, sem):
        off = s * rows_n
        for o, c in chunks:
            pltpu.async_copy(w_out.at[ia_v.at[pl.ds(off + o, c)]],
                             rbuf.at[pl.ds(o, c)], sem)

    def drain(rbuf, sem):
        for o, c in chunks:
            pltpu.make_async_copy(w_out.at[ia_v.at[pl.ds(o, c)]],
                                  rbuf.at[pl.ds(o, c)], sem).wait()

    fire(0, rows0, sem0)
    for cp in xcps:
        cp.wait()

    def compute(s, rbuf):
        def elem(e, carry):
            b = s * EC + e
            xr = [x_v[b, pl.ds(k * LANES, LANES)] for k in range(nseg)]
            r0 = e * npe
            yr = [rbuf[r0, pl.ds(k * LANES, LANES)] for k in range(nseg)]
            acc = xr[0] * yr[0]
            for k in range(1, nseg):
                acc = acc + xr[k] * yr[k]
            scatter1(pos_v, b, jnp.sum(acc))
            for n_i in range(neg):
                nr = [rbuf[r0 + 1 + n_i, pl.ds(k * LANES, LANES)]
                      for k in range(nseg)]
                nacc = nr[0] * xr[0]
                for k in range(1, nseg):
                    nacc = nacc + nr[k] * xr[k]
                scatter1(negs_v, b * neg + n_i, jnp.sum(nacc))
            return carry

        lax.fori_loop(0, EC, elem, 0)

    def outer(t, carry):
        s0 = 2 * t
        fire(s0 + 1, rows1, sem1)
        drain(rows0, sem0)
        compute(s0, rows0)
        fire(jnp.minimum(s0 + 2, steps - 1), rows0, sem0)
        drain(rows1, sem1)
        compute(s0 + 1, rows1)
        return carry

    lax.fori_loop(0, steps // 2, outer, 0)
    drain(rows0, sem0)

    pltpu.sync_copy(pos_v, pos_out.at[pl.ds(base, pw)])
    pltpu.sync_copy(negs_v, negs_out.at[pl.ds(base * neg, pw * neg)])


def _tc_loss_body(pos_ref, neg_ref, out_ref):
    p = pos_ref[...]
    n = neg_ref[...]
    ls_p = jnp.minimum(p, 0.0) - jnp.log1p(jnp.exp(-jnp.abs(p)))
    ls_n = jnp.minimum(-n, 0.0) - jnp.log1p(jnp.exp(-jnp.abs(n)))
    out_ref[0, 0] = -(jnp.sum(ls_p) + jnp.sum(ls_n))


def kernel(positive_pairs, negative_samples, W_hid, W_out):
    batch, neg = negative_samples.shape
    vocab, emb = W_hid.shape
    pw = batch // NW
    npe = neg + 1

    ix = positive_pairs[:, 0]
    ia = jnp.concatenate(
        [positive_pairs[:, 1:2], negative_samples], axis=1).reshape(-1)

    mesh = plsc.VectorSubcoreMesh(
        core_axis_name="c", subcore_axis_name="s",
        num_cores=NC, num_subcores=NS)

    transpose_tables = pl.kernel(
        functools.partial(_tr_body, vocab, emb),
        out_type=(jax.ShapeDtypeStruct((vocab * emb,), jnp.float32),
                  jax.ShapeDtypeStruct((vocab * emb,), jnp.float32)),
        mesh=mesh,
        scratch_types=[
            pltpu.VMEM((emb, TB), jnp.float32),
            pltpu.VMEM((emb, TB), jnp.float32),
            pltpu.VMEM((emb, TB), jnp.float32),
            pltpu.VMEM((emb, TB), jnp.float32),
            pltpu.VMEM((TB * emb,), jnp.float32),
            pltpu.VMEM((TB * emb,), jnp.float32),
            pltpu.VMEM((TB * emb,), jnp.float32),
            pltpu.VMEM((TB * emb,), jnp.float32),
            pltpu.SemaphoreType.DMA,
            pltpu.SemaphoreType.DMA,
            pltpu.SemaphoreType.DMA,
            pltpu.SemaphoreType.DMA,
        ],
        compiler_params=pltpu.CompilerParams(
            needs_layout_passes=False, use_tc_tiling_on_sc=True,
            disable_bounds_checks=True),
    )
    dh_flat, do_flat = transpose_tables(W_hid.T, W_out.T)
    w_hid_d = dh_flat.reshape(vocab, emb)
    w_out_d = do_flat.reshape(vocab, emb)
    sc_scores = pl.kernel(
        functools.partial(_sc_body, pw, neg, emb),
        out_type=(jax.ShapeDtypeStruct((batch,), jnp.float32),
                  jax.ShapeDtypeStruct((batch * neg,), jnp.float32)),
        mesh=mesh,
        scratch_types=[
            pltpu.VMEM((pw,), jnp.int32),
            pltpu.VMEM((pw * npe,), jnp.int32),
            pltpu.VMEM((pw, emb), jnp.float32),
            pltpu.VMEM((EC * npe, emb), jnp.float32),
            pltpu.VMEM((EC * npe, emb), jnp.float32),
            pltpu.VMEM((pw,), jnp.float32),
            pltpu.VMEM((pw * neg,), jnp.float32),
            pltpu.SemaphoreType.DMA,
            pltpu.SemaphoreType.DMA,
            pltpu.SemaphoreType.DMA,
        ],
        compiler_params=pltpu.CompilerParams(
            needs_layout_passes=False, use_tc_tiling_on_sc=False),
    )
    pos_s, neg_s = sc_scores(ix, ia, w_hid_d, w_out_d)

    pos2 = pos_s.reshape(batch // 128, 128)
    neg2 = neg_s.reshape(batch * neg // 128, 128)
    loss = pl.pallas_call(
        _tc_loss_body,
        out_shape=jax.ShapeDtypeStruct((1, 1), jnp.float32),
        out_specs=pl.BlockSpec(memory_space=pltpu.SMEM),
    )(pos2, neg2)
    return loss[0, 0]


# speedup vs baseline: 4.2410x; 1.6559x over previous
import functools

import jax
import jax.numpy as jnp
from jax import lax
from jax.experimental import pallas as pl
from jax.experimental.pallas import tpu as pltpu
from jax.experimental.pallas import tpu_sc as plsc

NC = 2
NS = 16
LANES = 16
NW = NC * NS

EC = 16
IDX_CHUNK = 128

TB = 128


def _tr_body(vocab, emb, wt_hid, wt_out, dh_flat, do_flat,
             ih0, io0, ih1, io1, obh0, obo0, obh1, obo1,
             sem0, sem1, semw0, semw1):
    nfull = vocab // TB
    per = nfull // NW + 1
    if per % 2:
        per += 1
    wid = lax.axis_index("s") * NC + lax.axis_index("c")
    blk = TB * emb

    def fire(b, ih, io, sem):
        pltpu.async_copy(wt_hid.at[:, pl.ds(b * TB, TB)], ih, sem)
        pltpu.async_copy(wt_out.at[:, pl.ds(b * TB, TB)], io, sem)

    def drain(ih, io, sem):
        pltpu.make_async_copy(wt_hid.at[:, pl.ds(0, TB)], ih, sem).wait()
        pltpu.make_async_copy(wt_out.at[:, pl.ds(0, TB)], io, sem).wait()

    def transpose(ih, io, obh, obo):
        lanes = lax.iota(jnp.int32, 16)

        def grp(g, carry):
            cols = lanes + g * 16
            colse = cols * emb
            for d0 in range(0, emb, 16):
                rows = [((lanes + i) & 15) + d0 for i in range(16)]
                vh = [plsc.load_gather(ih, [r, cols]) for r in rows]
                vo = [plsc.load_gather(io, [r, cols]) for r in rows]
                for i in range(16):
                    plsc.store_scatter(obh, [colse + rows[i]], vh[i])
                    plsc.store_scatter(obo, [colse + rows[i]], vo[i])
            return carry

        lax.fori_loop(0, TB // 16, grp, 0)

    def write(b, obh, obo, semw, n=None):
        n = blk if n is None else n
        pltpu.async_copy(obh.at[pl.ds(0, n)], dh_flat.at[pl.ds(b * blk, n)],
                         semw)
        pltpu.async_copy(obo.at[pl.ds(0, n)], do_flat.at[pl.ds(b * blk, n)],
                         semw)

    def wait_write(obh, obo, semw, n=None):
        n = blk if n is None else n
        pltpu.make_async_copy(obh.at[pl.ds(0, n)], dh_flat.at[pl.ds(0, n)],
                              semw).wait()
        pltpu.make_async_copy(obo.at[pl.ds(0, n)], do_flat.at[pl.ds(0, n)],
                              semw).wait()

    def bidx(i):
        return jnp.minimum(wid + NW * i, nfull - 1)

    fire(bidx(0), ih0, io0, sem0)
    fire(bidx(1), ih1, io1, sem1)
    drain(ih0, io0, sem0)
    transpose(ih0, io0, obh0, obo0)
    write(bidx(0), obh0, obo0, semw0)
    fire(bidx(2), ih0, io0, sem0)
    drain(ih1, io1, sem1)
    transpose(ih1, io1, obh1, obo1)
    write(bidx(1), obh1, obo1, semw1)

    def outer(p, carry):
        i0 = 2 * p
        fire(bidx(i0 + 1), ih1, io1, sem1)
        drain(ih0, io0, sem0)
        wait_write(obh0, obo0, semw0)
        transpose(ih0, io0, obh0, obo0)
        write(bidx(i0), obh0, obo0, semw0)
        fire(bidx(i0 + 2), ih0, io0, sem0)
        drain(ih1, io1, sem1)
        wait_write(obh1, obo1, semw1)
        transpose(ih1, io1, obh1, obo1)
        write(bidx(i0 + 1), obh1, obo1, semw1)
        return carry

    lax.fori_loop(1, per // 2, outer, 0)
    drain(ih0, io0, sem0)
    wait_write(obh0, obo0, semw0)
    wait_write(obh1, obo1, semw1)

    rem = vocab - nfull * TB
    if rem:
        @pl.when(wid == 0)
        def _tail():
            b0 = jnp.int32(nfull) + jnp.int32(0)
            fire(b0, ih0, io0, sem0)
            drain(ih0, io0, sem0)
            transpose(ih0, io0, obh0, obo0)
            write(b0, obh0, obo0, semw0, n=rem * emb)
            wait_write(obh0, obo0, semw0, n=rem * emb)


def _sc_body(pw, neg, emb, ix_hbm, ia_hbm, w_hid, w_out, pos_out, negs_out,
             xi_v, ia_v, x_v, rows0, rows1, pos_v, negs_v, semx, sem0, sem1):
    npe = neg + 1
    nseg = emb // LANES
    steps = pw // EC
    rows_n = EC * npe
    chunks = [(o, min(IDX_CHUNK, rows_n - o)) for o in range(0, rows_n, IDX_CHUNK)]
    wid = lax.axis_index("s") * NC + lax.axis_index("c")
    base = wid * pw
    lane0 = lax.iota(jnp.int32, 16) == 0

    def scatter1(ref, pos_i, val):
        plsc.store_scatter(ref, [jnp.broadcast_to(pos_i, (16,))],
                           jnp.broadcast_to(val, (16,)), mask=lane0)

    pltpu.sync_copy(ix_hbm.at[pl.ds(base, pw)], xi_v)
    pltpu.sync_copy(ia_hbm.at[pl.ds(base * npe, pw * npe)], ia_v)

    xcps = [pltpu.make_async_copy(
        w_hid.at[xi_v.at[pl.ds(j * IDX_CHUNK, IDX_CHUNK)]],
        x_v.at[pl.ds(j * IDX_CHUNK, IDX_CHUNK)], semx)
        for j in range(pw // IDX_CHUNK)]
    for cp in xcps:
        cp.start()

    def fire(s, rbuf, sem):
        off = s * rows_n
        for o, c in chunks:
            pltpu.async_copy(w_out.at[ia_v.at[pl.ds(off + o, c)]],
                             rbuf.at[pl.ds(o, c)], sem)

    def drain(rbuf, sem):
        for o, c in chunks:
            pltpu.make_async_copy(w_out.at[ia_v.at[pl.ds(o, c)]],
                                  rbuf.at[pl.ds(o, c)], sem).wait()

    fire(0, rows0, sem0)
    for cp in xcps:
        cp.wait()

    def compute(s, rbuf):
        def elem(e, carry):
            b = s * EC + e
            xr = [x_v[b, pl.ds(k * LANES, LANES)] for k in range(nseg)]
            r0 = e * npe
            yr = [rbuf[r0, pl.ds(k * LANES, LANES)] for k in range(nseg)]
            acc = xr[0] * yr[0]
            for k in range(1, nseg):
                acc = acc + xr[k] * yr[k]
            scatter1(pos_v, b, jnp.sum(acc))
            for n_i in range(neg):
                nr = [rbuf[r0 + 1 + n_i, pl.ds(k * LANES, LANES)]
                      for k in range(nseg)]
                nacc = nr[0] * xr[0]
                for k in range(1, nseg):
                    nacc = nacc + nr[k] * xr[k]
                scatter1(negs_v, b * neg + n_i, jnp.sum(nacc))
            return carry

        lax.fori_loop(0, EC, elem, 0)

    def outer(t, carry):
        s0 = 2 * t
        fire(s0 + 1, rows1, sem1)
        drain(rows0, sem0)
        compute(s0, rows0)
        fire(jnp.minimum(s0 + 2, steps - 1), rows0, sem0)
        drain(rows1, sem1)
        compute(s0 + 1, rows1)
        return carry

    lax.fori_loop(0, steps // 2, outer, 0)
    drain(rows0, sem0)

    pltpu.sync_copy(pos_v, pos_out.at[pl.ds(base, pw)])
    pltpu.sync_copy(negs_v, negs_out.at[pl.ds(base * neg, pw * neg)])


def _tc_loss_body(pos_ref, neg_ref, out_ref):
    p = pos_ref[...]
    n = neg_ref[...]
    ls_p = jnp.minimum(p, 0.0) - jnp.log1p(jnp.exp(-jnp.abs(p)))
    ls_n = jnp.minimum(-n, 0.0) - jnp.log1p(jnp.exp(-jnp.abs(n)))
    out_ref[0, 0] = -(jnp.sum(ls_p) + jnp.sum(ls_n))


def kernel(positive_pairs, negative_samples, W_hid, W_out):
    batch, neg = negative_samples.shape
    vocab, emb = W_hid.shape
    pw = batch // NW
    npe = neg + 1

    ix = positive_pairs[:, 0]
    ia = jnp.concatenate(
        [positive_pairs[:, 1:2], negative_samples], axis=1).reshape(-1)

    mesh = plsc.VectorSubcoreMesh(
        core_axis_name="c", subcore_axis_name="s",
        num_cores=NC, num_subcores=NS)

    transpose_tables = pl.kernel(
        functools.partial(_tr_body, vocab, emb),
        out_type=(jax.ShapeDtypeStruct((vocab * emb,), jnp.float32),
                  jax.ShapeDtypeStruct((vocab * emb,), jnp.float32)),
        mesh=mesh,
        scratch_types=[
            pltpu.VMEM((emb, TB), jnp.float32),
            pltpu.VMEM((emb, TB), jnp.float32),
            pltpu.VMEM((emb, TB), jnp.float32),
            pltpu.VMEM((emb, TB), jnp.float32),
            pltpu.VMEM((TB * emb,), jnp.float32),
            pltpu.VMEM((TB * emb,), jnp.float32),
            pltpu.VMEM((TB * emb,), jnp.float32),
            pltpu.VMEM((TB * emb,), jnp.float32),
            pltpu.SemaphoreType.DMA,
            pltpu.SemaphoreType.DMA,
            pltpu.SemaphoreType.DMA,
            pltpu.SemaphoreType.DMA,
        ],
        compiler_params=pltpu.CompilerParams(
            needs_layout_passes=False, use_tc_tiling_on_sc=True,
            disable_bounds_checks=True),
    )
    dh_flat, do_flat = transpose_tables(W_hid.T, W_out.T)
    w_hid_d = dh_flat.reshape(vocab, emb)
    w_out_d = do_flat.reshape(vocab, emb)
    sc_scores = pl.kernel(
        functools.partial(_sc_body, pw, neg, emb),
        out_type=(jax.ShapeDtypeStruct((batch,), jnp.float32),
                  jax.ShapeDtypeStruct((batch * neg,), jnp.float32)),
        mesh=mesh,
        scratch_types=[
            pltpu.VMEM((pw,), jnp.int32),
            pltpu.VMEM((pw * npe,), jnp.int32),
            pltpu.VMEM((pw, emb), jnp.float32),
            pltpu.VMEM((EC * npe, emb), jnp.float32),
            pltpu.VMEM((EC * npe, emb), jnp.float32),
            pltpu.VMEM((pw,), jnp.float32),
            pltpu.VMEM((pw * neg,), jnp.float32),
            pltpu.SemaphoreType.DMA,
            pltpu.SemaphoreType.DMA,
            pltpu.SemaphoreType.DMA,
        ],
        compiler_params=pltpu.CompilerParams(
            needs_layout_passes=False, use_tc_tiling_on_sc=False),
    )
    pos_s, neg_s = sc_scores(ix, ia, w_hid_d, w_out_d)

    pos2 = pos_s.reshape(batch // 128, 128)
    neg2 = neg_s.reshape(batch * neg // 128, 128)
    loss = pl.pallas_call(
        _tc_loss_body,
        out_shape=jax.ShapeDtypeStruct((1, 1), jnp.float32),
        out_specs=pl.BlockSpec(memory_space=pltpu.SMEM),
    )(pos2, neg2)
    return loss[0, 0]
